# baseline (speedup 1.0000x reference)
import jax
import jax.numpy as jnp
from jax import lax
from jax.experimental import pallas as pl

_B, _N, _S, _D1, _D2 = 16, 4096, 1024, 256, 512
_CIN = _D1 + _D2
_M0, _M1 = 512, 512
_BLK = 1024
_NB = _N // _BLK
_BLK2 = 1024
_NB2 = _N // _BLK2


def _stage0_body(p2_ref, w1a_ref, o_ref):
    o_ref[0] = jnp.dot(w1a_ref[...], p2_ref[0].astype(jnp.bfloat16),
                       preferred_element_type=jnp.float32).astype(jnp.bfloat16)


def _stage1_body(x2t_ref, x1_ref, wp2_ref, p1_ref, w1b_ref,
                 h_ref, sum_ref, sq_ref):
    x2t = x2t_ref[0]
    x1b = x1_ref[0]
    n2 = 0.25 * jnp.sum(x2t * x2t, axis=1, keepdims=True)
    d0 = jnp.dot(x2t, x1b, preferred_element_type=jnp.float32) + n2
    n1 = jnp.sum(x1b * x1b, axis=0, keepdims=True)

    m1 = jnp.min(d0, axis=0, keepdims=True)
    m2 = jnp.min(jnp.where(d0 <= m1, jnp.float32(jnp.inf), d0), axis=0,
                 keepdims=True)
    m3 = jnp.min(jnp.where(d0 <= m2, jnp.float32(jnp.inf), d0), axis=0,
                 keepdims=True)
    c = n1 + 1e-8
    inv_rs = 1.0 / (1.0 / (m1 + c) + 1.0 / (m2 + c) + 1.0 / (m3 + c))
    wgt = jnp.where(d0 <= m3, inv_rs / (d0 + c), 0.0).astype(jnp.bfloat16)

    h = jnp.dot(wp2_ref[0], wgt, preferred_element_type=jnp.float32)
    h = h + jnp.dot(w1b_ref[...], p1_ref[0].astype(jnp.bfloat16),
                    preferred_element_type=jnp.float32)
    h_ref[0] = h.astype(jnp.bfloat16)

    @pl.when((pl.program_id(0) == 0) & (pl.program_id(1) == 0))
    def _init():
        sum_ref[...] = jnp.zeros_like(sum_ref)
        sq_ref[...] = jnp.zeros_like(sq_ref)

    ones = jnp.ones((_BLK, 1), jnp.float32)
    sum_ref[...] += jnp.dot(h, ones, preferred_element_type=jnp.float32)
    sq_ref[...] += jnp.dot(h * h, ones, preferred_element_type=jnp.float32)


def _stage2_body(h_ref, sum_ref, sq_ref, g1_ref, be_ref, w2_ref, b2_ref,
                 out_ref):
    inv_cnt = 1.0 / (_B * _N)
    mean = sum_ref[...] * inv_cnt
    var = sq_ref[...] * inv_cnt - mean * mean
    scale = g1_ref[...] * lax.rsqrt(var + 1e-5)
    shift = be_ref[...] - mean * scale
    g = jnp.maximum(h_ref[0].astype(jnp.float32) * scale + shift, 0.0)
    o = jnp.dot(w2_ref[...], g.astype(jnp.bfloat16),
                preferred_element_type=jnp.float32) + b2_ref[...]
    out_ref[0] = jnp.maximum(o, 0.0)


def kernel(xyz1, xyz2, points1, points2, W1, b1, gamma1, beta1, W2, b2):
    del b1
    w1a = W1[:, :_D2].astype(jnp.bfloat16)
    w1b = W1[:, _D2:].astype(jnp.bfloat16)

    x2t = jnp.transpose(xyz2, (0, 2, 1)) * (-2.0)
    wp2 = pl.pallas_call(
        _stage0_body,
        grid=(_B,),
        in_specs=[
            pl.BlockSpec((1, _D2, _S), lambda b: (b, 0, 0)),
            pl.BlockSpec((_M0, _D2), lambda b: (0, 0)),
        ],
        out_specs=pl.BlockSpec((1, _M0, _S), lambda b: (b, 0, 0)),
        out_shape=jax.ShapeDtypeStruct((_B, _M0, _S), jnp.bfloat16),
    )(points2, w1a)

    h, hsum, hsq = pl.pallas_call(
        _stage1_body,
        grid=(_B, _NB),
        in_specs=[
            pl.BlockSpec((1, _S, 3), lambda b, n: (b, 0, 0)),
            pl.BlockSpec((1, 3, _BLK), lambda b, n: (b, 0, n)),
            pl.BlockSpec((1, _M0, _S), lambda b, n: (b, 0, 0)),
            pl.BlockSpec((1, _D1, _BLK), lambda b, n: (b, 0, n)),
            pl.BlockSpec((_M0, _D1), lambda b, n: (0, 0)),
        ],
        out_specs=[
            pl.BlockSpec((1, _M0, _BLK), lambda b, n: (b, 0, n)),
            pl.BlockSpec((_M0, 1), lambda b, n: (0, 0)),
            pl.BlockSpec((_M0, 1), lambda b, n: (0, 0)),
        ],
        out_shape=[
            jax.ShapeDtypeStruct((_B, _M0, _N), jnp.bfloat16),
            jax.ShapeDtypeStruct((_M0, 1), jnp.float32),
            jax.ShapeDtypeStruct((_M0, 1), jnp.float32),
        ],
    )(x2t, xyz1, wp2, points1, w1b)

    out = pl.pallas_call(
        _stage2_body,
        grid=(_B, _NB2),
        in_specs=[
            pl.BlockSpec((1, _M0, _BLK2), lambda b, n: (b, 0, n)),
            pl.BlockSpec((_M0, 1), lambda b, n: (0, 0)),
            pl.BlockSpec((_M0, 1), lambda b, n: (0, 0)),
            pl.BlockSpec((_M0, 1), lambda b, n: (0, 0)),
            pl.BlockSpec((_M0, 1), lambda b, n: (0, 0)),
            pl.BlockSpec((_M1, _M0), lambda b, n: (0, 0)),
            pl.BlockSpec((_M1, 1), lambda b, n: (0, 0)),
        ],
        out_specs=pl.BlockSpec((1, _M1, _BLK2), lambda b, n: (b, 0, n)),
        out_shape=jax.ShapeDtypeStruct((_B, _M1, _N), jnp.float32),
    )(h, hsum, hsq, gamma1[:, None], beta1[:, None],
      W2.astype(jnp.bfloat16), b2[:, None])
    return out


# speedup vs baseline: 1.0277x; 1.0277x over previous
import jax
import jax.numpy as jnp
from jax import lax
from jax.experimental import pallas as pl
from jax.experimental.pallas import tpu as pltpu

_B, _N, _S, _D1, _D2 = 16, 4096, 1024, 256, 512
_CIN = _D1 + _D2
_M0, _M1 = 512, 512
_BLK = 1024
_NB = _N // _BLK
_BLK2 = 1024
_NB2 = _N // _BLK2


def _stage1_body(x2t_ref, x1_ref, p2_ref, p1_ref, w1a_ref, w1b_ref,
                 h_ref, sum_ref, sq_ref, wp2_ref):
    @pl.when(pl.program_id(1) == 0)
    def _fold():
        wp2_ref[...] = jnp.dot(
            w1a_ref[...], p2_ref[0].astype(jnp.bfloat16),
            preferred_element_type=jnp.float32).astype(jnp.bfloat16)

    x2t = x2t_ref[0]
    x1b = x1_ref[0]
    n2 = 0.25 * jnp.sum(x2t * x2t, axis=1, keepdims=True)
    d0 = jnp.dot(x2t, x1b, preferred_element_type=jnp.float32) + n2
    n1 = jnp.sum(x1b * x1b, axis=0, keepdims=True)

    m1 = jnp.min(d0, axis=0, keepdims=True)
    m2 = jnp.min(jnp.where(d0 <= m1, jnp.float32(jnp.inf), d0), axis=0,
                 keepdims=True)
    m3 = jnp.min(jnp.where(d0 <= m2, jnp.float32(jnp.inf), d0), axis=0,
                 keepdims=True)
    c = n1 + 1e-8
    inv_rs = 1.0 / (1.0 / (m1 + c) + 1.0 / (m2 + c) + 1.0 / (m3 + c))
    wgt = jnp.where(d0 <= m3, inv_rs / (d0 + c), 0.0).astype(jnp.bfloat16)

    h = jnp.dot(wp2_ref[...], wgt, preferred_element_type=jnp.float32)
    h = h + jnp.dot(w1b_ref[...], p1_ref[0].astype(jnp.bfloat16),
                    preferred_element_type=jnp.float32)
    h_ref[0] = h.astype(jnp.bfloat16)

    @pl.when((pl.program_id(0) == 0) & (pl.program_id(1) == 0))
    def _init():
        sum_ref[...] = jnp.zeros_like(sum_ref)
        sq_ref[...] = jnp.zeros_like(sq_ref)

    ones = jnp.ones((_BLK, 1), jnp.float32)
    sum_ref[...] += jnp.dot(h, ones, preferred_element_type=jnp.float32)
    sq_ref[...] += jnp.dot(h * h, ones, preferred_element_type=jnp.float32)


def _stage2_body(h_ref, sum_ref, sq_ref, g1_ref, be_ref, w2_ref, b2_ref,
                 out_ref):
    inv_cnt = 1.0 / (_B * _N)
    mean = sum_ref[...] * inv_cnt
    var = sq_ref[...] * inv_cnt - mean * mean
    scale = g1_ref[...] * lax.rsqrt(var + 1e-5)
    shift = be_ref[...] - mean * scale
    g = jnp.maximum(h_ref[0].astype(jnp.float32) * scale + shift, 0.0)
    o = jnp.dot(w2_ref[...], g.astype(jnp.bfloat16),
                preferred_element_type=jnp.float32) + b2_ref[...]
    out_ref[0] = jnp.maximum(o, 0.0)


def kernel(xyz1, xyz2, points1, points2, W1, b1, gamma1, beta1, W2, b2):
    del b1
    w1a = W1[:, :_D2].astype(jnp.bfloat16)
    w1b = W1[:, _D2:].astype(jnp.bfloat16)

    x2t = jnp.transpose(xyz2, (0, 2, 1)) * (-2.0)

    h, hsum, hsq = pl.pallas_call(
        _stage1_body,
        grid=(_B, _NB),
        in_specs=[
            pl.BlockSpec((1, _S, 3), lambda b, n: (b, 0, 0)),
            pl.BlockSpec((1, 3, _BLK), lambda b, n: (b, 0, n)),
            pl.BlockSpec((1, _D2, _S), lambda b, n: (b, 0, 0)),
            pl.BlockSpec((1, _D1, _BLK), lambda b, n: (b, 0, n)),
            pl.BlockSpec((_M0, _D2), lambda b, n: (0, 0)),
            pl.BlockSpec((_M0, _D1), lambda b, n: (0, 0)),
        ],
        out_specs=[
            pl.BlockSpec((1, _M0, _BLK), lambda b, n: (b, 0, n)),
            pl.BlockSpec((_M0, 1), lambda b, n: (0, 0)),
            pl.BlockSpec((_M0, 1), lambda b, n: (0, 0)),
        ],
        out_shape=[
            jax.ShapeDtypeStruct((_B, _M0, _N), jnp.bfloat16),
            jax.ShapeDtypeStruct((_M0, 1), jnp.float32),
            jax.ShapeDtypeStruct((_M0, 1), jnp.float32),
        ],
        scratch_shapes=[pltpu.VMEM((_M0, _S), jnp.bfloat16)],
    )(x2t, xyz1, points2, points1, w1a, w1b)

    out = pl.pallas_call(
        _stage2_body,
        grid=(_B, _NB2),
        in_specs=[
            pl.BlockSpec((1, _M0, _BLK2), lambda b, n: (b, 0, n)),
            pl.BlockSpec((_M0, 1), lambda b, n: (0, 0)),
            pl.BlockSpec((_M0, 1), lambda b, n: (0, 0)),
            pl.BlockSpec((_M0, 1), lambda b, n: (0, 0)),
            pl.BlockSpec((_M0, 1), lambda b, n: (0, 0)),
            pl.BlockSpec((_M1, _M0), lambda b, n: (0, 0)),
            pl.BlockSpec((_M1, 1), lambda b, n: (0, 0)),
        ],
        out_specs=pl.BlockSpec((1, _M1, _BLK2), lambda b, n: (b, 0, n)),
        out_shape=jax.ShapeDtypeStruct((_B, _M1, _N), jnp.float32),
    )(h, hsum, hsq, gamma1[:, None], beta1[:, None],
      W2.astype(jnp.bfloat16), b2[:, None])
    return out
